# baseline (speedup 1.0000x reference)
import functools

import jax
import jax.numpy as jnp
from jax import lax
from jax.experimental import pallas as pl
from jax.experimental.pallas import tpu as pltpu
from jax.experimental.pallas import tpu_sc as plsc

B, S, HID, MAXP = 4, 4096, 1024, 4096
EPS = 1e-12
N = B * S

LANES = 16
VPR = HID // LANES
CH = 16
NC, NS = 2, 16
NW = NC * NS
ROWS_PER_W = N // NW
NCHUNKS = ROWS_PER_W // CH


def _rsqrt(y):
    i = lax.bitcast_convert_type(y, jnp.int32)
    i = jnp.int32(0x5F3759DF) - lax.shift_right_arithmetic(i, 1)
    r = lax.bitcast_convert_type(i, jnp.float32)
    half = 0.5 * y
    for _ in range(3):
        r = r * (1.5 - half * r * r)
    return r


def _lane_sum(vec):
    vals = [vec[i] for i in range(LANES)]
    while len(vals) > 1:
        vals = [a + b for a, b in zip(vals[0::2], vals[1::2])]
    return vals[0]


def _emb_ln_body(x_hbm, ids_hbm, tab_hbm, g_hbm, b_hbm, out_hbm,
                 idx_v, x_v, t_v, o_v, g_v, b_v, mean_s, rstd_s,
                 sem_g, sem_x, sem_o):
    wid = lax.axis_index("s") * NC + lax.axis_index("c")
    base_row = wid * ROWS_PER_W

    pltpu.sync_copy(g_hbm, g_v)
    pltpu.sync_copy(b_hbm, b_v)

    zeros = jnp.zeros((LANES,), jnp.float32)

    def chunk(c, _):
        row0 = base_row + c * CH
        pltpu.sync_copy(ids_hbm.at[pl.ds(row0, CH)], idx_v)
        gcp = pltpu.async_copy(tab_hbm.at[idx_v], t_v, sem_g)
        xcp = pltpu.async_copy(x_hbm.at[pl.ds(row0, CH)], x_v, sem_x)
        gcp.wait()
        xcp.wait()

        def row_stats(r, _):
            def acc(j, carry):
                s, q = carry
                sl = pl.ds(j * LANES, LANES)
                v = x_v[r, sl] + t_v[r, sl]
                x_v[r, sl] = v
                return s + v, q + v * v

            s, q = lax.fori_loop(0, VPR, acc, (zeros, zeros))
            mean = _lane_sum(s) * (1.0 / HID)
            msq = _lane_sum(q) * (1.0 / HID)
            var = msq - mean * mean
            mean_s[r] = mean
            rstd_s[r] = _rsqrt(var + EPS)
            return 0

        lax.fori_loop(0, CH, row_stats, 0)

        def col(j, _):
            sl = pl.ds(j * LANES, LANES)
            g = g_v[sl]
            b = b_v[sl]

            def row_norm(r, _):
                v = x_v[r, sl]
                o_v[r, sl] = (v - mean_s[r]) * rstd_s[r] * g + b
                return 0

            lax.fori_loop(0, CH, row_norm, 0)
            return 0

        lax.fori_loop(0, VPR, col, 0)

        pltpu.async_copy(o_v, out_hbm.at[pl.ds(row0, CH)], sem_o).wait()
        return 0

    lax.fori_loop(0, NCHUNKS, chunk, 0)


@jax.jit
def _emb_ln(x, ids, tab, gamma, beta):
    mesh = plsc.VectorSubcoreMesh(core_axis_name="c", subcore_axis_name="s")
    return pl.kernel(
        _emb_ln_body,
        out_type=jax.ShapeDtypeStruct((N, HID), jnp.float32),
        mesh=mesh,
        scratch_types=[
            pltpu.VMEM((CH,), jnp.int32),
            pltpu.VMEM((CH, HID), jnp.float32),
            pltpu.VMEM((CH, HID), jnp.float32),
            pltpu.VMEM((CH, HID), jnp.float32),
            pltpu.VMEM((HID,), jnp.float32),
            pltpu.VMEM((HID,), jnp.float32),
            pltpu.SMEM((CH,), jnp.float32),
            pltpu.SMEM((CH,), jnp.float32),
            pltpu.SemaphoreType.DMA,
            pltpu.SemaphoreType.DMA,
            pltpu.SemaphoreType.DMA,
        ],
    )(x, ids, tab, gamma, beta)


def kernel(input_embeds, position_ids, pos_table, gamma, beta):
    x = input_embeds.reshape(N, HID)
    ids = position_ids.reshape(N)
    out = _emb_ln(x, ids, pos_table, gamma, beta)
    return out.reshape(B, S, HID)


# speedup vs baseline: 1.4077x; 1.4077x over previous
import functools

import jax
import jax.numpy as jnp
from jax import lax
from jax.experimental import pallas as pl
from jax.experimental.pallas import tpu as pltpu
from jax.experimental.pallas import tpu_sc as plsc

B, S, HID, MAXP = 4, 4096, 1024, 4096
EPS = 1e-12
N = B * S

LANES = 16
VPR = HID // LANES
CH = 16
NC, NS = 2, 16
NW = NC * NS
ROWS_PER_W = N // NW
NCHUNKS = ROWS_PER_W // CH
NBUF = 2


def _rsqrt(y):
    i = lax.bitcast_convert_type(y, jnp.int32)
    i = jnp.int32(0x5F3759DF) - lax.shift_right_arithmetic(i, 1)
    r = lax.bitcast_convert_type(i, jnp.float32)
    half = 0.5 * y
    for _ in range(3):
        r = r * (1.5 - half * r * r)
    return r


def _lane_sum(vec):
    vals = [vec[i] for i in range(LANES)]
    while len(vals) > 1:
        vals = [a + b for a, b in zip(vals[0::2], vals[1::2])]
    return vals[0]


def _emb_ln_body(x_hbm, ids_hbm, tab_hbm, g_hbm, b_hbm, out_hbm,
                 idx0, idx1, x0, x1, t0, t1, o0, o1, g_v, b_v,
                 mean_s, rstd_s,
                 sg0, sg1, sx0, sx1, so0, so1):
    idx_v = [idx0, idx1]
    x_v = [x0, x1]
    t_v = [t0, t1]
    o_v = [o0, o1]
    sem_g = [sg0, sg1]
    sem_x = [sx0, sx1]
    sem_o = [so0, so1]

    wid = lax.axis_index("s") * NC + lax.axis_index("c")
    base_row = wid * ROWS_PER_W

    pltpu.sync_copy(g_hbm, g_v)
    pltpu.sync_copy(b_hbm, b_v)

    zeros = jnp.zeros((LANES,), jnp.float32)

    def start_loads(c, bslot):
        row0 = base_row + c * CH
        pltpu.sync_copy(ids_hbm.at[pl.ds(row0, CH)], idx_v[bslot])
        pltpu.async_copy(tab_hbm.at[idx_v[bslot]], t_v[bslot], sem_g[bslot])
        pltpu.async_copy(x_hbm.at[pl.ds(row0, CH)], x_v[bslot], sem_x[bslot])

    def compute(bslot):
        xb, tb, ob = x_v[bslot], t_v[bslot], o_v[bslot]

        def row_stats(r, _):
            def acc(j, carry):
                s, q = carry
                sl = pl.ds(j * LANES, LANES)
                v = xb[r, sl] + tb[r, sl]
                xb[r, sl] = v
                return s + v, q + v * v

            s, q = lax.fori_loop(0, VPR, acc, (zeros, zeros), unroll=8)
            mean = _lane_sum(s) * (1.0 / HID)
            msq = _lane_sum(q) * (1.0 / HID)
            var = msq - mean * mean
            mean_s[r] = mean
            rstd_s[r] = _rsqrt(var + EPS)
            return 0

        lax.fori_loop(0, CH, row_stats, 0)

        def col(j, _):
            sl = pl.ds(j * LANES, LANES)
            g = g_v[sl]
            b = b_v[sl]

            def row_norm(r, _):
                v = xb[r, sl]
                ob[r, sl] = (v - mean_s[r]) * rstd_s[r] * g + b
                return 0

            lax.fori_loop(0, CH, row_norm, 0, unroll=8)
            return 0

        lax.fori_loop(0, VPR, col, 0)

    for b in range(NBUF):
        start_loads(b, b)

    def body(c0, _):
        for b in range(NBUF):
            c = c0 + b
            pltpu.make_async_copy(tab_hbm.at[idx_v[b]], t_v[b], sem_g[b]).wait()
            row0 = base_row + c * CH
            pltpu.make_async_copy(x_hbm.at[pl.ds(row0, CH)], x_v[b],
                                  sem_x[b]).wait()

            @pl.when(c >= NBUF)
            def _():
                prev0 = base_row + (c - NBUF) * CH
                pltpu.make_async_copy(o_v[b], out_hbm.at[pl.ds(prev0, CH)],
                                      sem_o[b]).wait()

            compute(b)
            pltpu.async_copy(o_v[b], out_hbm.at[pl.ds(row0, CH)], sem_o[b])

            @pl.when(c + NBUF < NCHUNKS)
            def _():
                start_loads(c + NBUF, b)

        return 0

    lax.fori_loop(0, NCHUNKS // NBUF, lambda i, _: body(i * NBUF, _), 0)

    for b in range(NBUF):
        c = NCHUNKS - NBUF + b
        row0 = base_row + c * CH
        pltpu.make_async_copy(o_v[b], out_hbm.at[pl.ds(row0, CH)],
                              sem_o[b]).wait()


@jax.jit
def _emb_ln(x, ids, tab, gamma, beta):
    mesh = plsc.VectorSubcoreMesh(core_axis_name="c", subcore_axis_name="s")
    return pl.kernel(
        _emb_ln_body,
        out_type=jax.ShapeDtypeStruct((N, HID), jnp.float32),
        mesh=mesh,
        scratch_types=[
            pltpu.VMEM((CH,), jnp.int32),
            pltpu.VMEM((CH,), jnp.int32),
            pltpu.VMEM((CH, HID), jnp.float32),
            pltpu.VMEM((CH, HID), jnp.float32),
            pltpu.VMEM((CH, HID), jnp.float32),
            pltpu.VMEM((CH, HID), jnp.float32),
            pltpu.VMEM((CH, HID), jnp.float32),
            pltpu.VMEM((CH, HID), jnp.float32),
            pltpu.VMEM((HID,), jnp.float32),
            pltpu.VMEM((HID,), jnp.float32),
            pltpu.SMEM((CH,), jnp.float32),
            pltpu.SMEM((CH,), jnp.float32),
            pltpu.SemaphoreType.DMA,
            pltpu.SemaphoreType.DMA,
            pltpu.SemaphoreType.DMA,
            pltpu.SemaphoreType.DMA,
            pltpu.SemaphoreType.DMA,
            pltpu.SemaphoreType.DMA,
        ],
    )(x, ids, tab, gamma, beta)


def kernel(input_embeds, position_ids, pos_table, gamma, beta):
    x = input_embeds.reshape(N, HID)
    ids = position_ids.reshape(N)
    out = _emb_ln(x, ids, pos_table, gamma, beta)
    return out.reshape(B, S, HID)


# speedup vs baseline: 1.4802x; 1.0515x over previous
import functools

import jax
import jax.numpy as jnp
from jax import lax
from jax.experimental import pallas as pl
from jax.experimental.pallas import tpu as pltpu
from jax.experimental.pallas import tpu_sc as plsc

B, S, HID, MAXP = 4, 4096, 1024, 4096
EPS = 1e-12
N = B * S

LANES = 16
VPR = HID // LANES
CH = 16
NC, NS = 2, 16
NW = NC * NS
ROWS_PER_W = N // NW
NCHUNKS = ROWS_PER_W // CH
NBUF = 2


def _rsqrt(y):
    i = lax.bitcast_convert_type(y, jnp.int32)
    i = jnp.int32(0x5F3759DF) - lax.shift_right_arithmetic(i, 1)
    r = lax.bitcast_convert_type(i, jnp.float32)
    half = 0.5 * y
    for _ in range(3):
        r = r * (1.5 - half * r * r)
    return r


def _emb_ln_body(x_hbm, ids_hbm, tab_hbm, g_hbm, b_hbm, out_hbm,
                 idx0, idx1, x0, x1, t0, t1, o0, o1, g_v, b_v, stage_v,
                 mean_s, rstd_s,
                 sg0, sg1, sx0, sx1, so0, so1):
    idx_v = [idx0, idx1]
    x_v = [x0, x1]
    t_v = [t0, t1]
    o_v = [o0, o1]
    sem_g = [sg0, sg1]
    sem_x = [sx0, sx1]
    sem_o = [so0, so1]

    wid = lax.axis_index("s") * NC + lax.axis_index("c")
    base_row = wid * ROWS_PER_W

    pltpu.sync_copy(g_hbm, g_v)
    pltpu.sync_copy(b_hbm, b_v)

    zeros = jnp.zeros((LANES,), jnp.float32)

    def start_loads(c, bslot):
        row0 = base_row + c * CH
        pltpu.sync_copy(ids_hbm.at[pl.ds(row0, CH)], idx_v[bslot])
        pltpu.async_copy(tab_hbm.at[idx_v[bslot]], t_v[bslot], sem_g[bslot])
        pltpu.async_copy(x_hbm.at[pl.ds(row0, CH)], x_v[bslot], sem_x[bslot])

    lane_iota = jnp.arange(LANES, dtype=jnp.int32)
    col_base = lane_iota * (2 * LANES)

    def compute(bslot):
        xb, tb, ob = x_v[bslot], t_v[bslot], o_v[bslot]

        def row_stats(r, _):
            def acc(j, carry):
                s, q = carry
                sl = pl.ds(j * LANES, LANES)
                v = xb[r, sl] + tb[r, sl]
                xb[r, sl] = v
                return s + v, q + v * v

            s, q = lax.fori_loop(0, VPR, acc, (zeros, zeros), unroll=8)
            stage_v[pl.ds(r * 2 * LANES, LANES)] = s
            stage_v[pl.ds(r * 2 * LANES + LANES, LANES)] = q
            return 0

        lax.fori_loop(0, CH, row_stats, 0)

        srow = zeros
        qrow = zeros
        for l in range(LANES):
            srow = srow + plsc.load_gather(stage_v, [col_base + l])
            qrow = qrow + plsc.load_gather(stage_v, [col_base + LANES + l])
        mean_v = srow * (1.0 / HID)
        var_v = qrow * (1.0 / HID) - mean_v * mean_v
        rstd_v = _rsqrt(var_v + EPS)
        for r in range(CH):
            mean_s[r] = mean_v[r]
            rstd_s[r] = rstd_v[r]

        def col(j, _):
            sl = pl.ds(j * LANES, LANES)
            g = g_v[sl]
            b = b_v[sl]

            def row_norm(r, _):
                v = xb[r, sl]
                ob[r, sl] = (v - mean_s[r]) * rstd_s[r] * g + b
                return 0

            lax.fori_loop(0, CH, row_norm, 0, unroll=8)
            return 0

        lax.fori_loop(0, VPR, col, 0)

    for b in range(NBUF):
        start_loads(b, b)

    def body(c0, _):
        for b in range(NBUF):
            c = c0 + b
            pltpu.make_async_copy(tab_hbm.at[idx_v[b]], t_v[b], sem_g[b]).wait()
            row0 = base_row + c * CH
            pltpu.make_async_copy(x_hbm.at[pl.ds(row0, CH)], x_v[b],
                                  sem_x[b]).wait()

            @pl.when(c >= NBUF)
            def _():
                prev0 = base_row + (c - NBUF) * CH
                pltpu.make_async_copy(o_v[b], out_hbm.at[pl.ds(prev0, CH)],
                                      sem_o[b]).wait()

            compute(b)
            pltpu.async_copy(o_v[b], out_hbm.at[pl.ds(row0, CH)], sem_o[b])

            @pl.when(c + NBUF < NCHUNKS)
            def _():
                start_loads(c + NBUF, b)

        return 0

    lax.fori_loop(0, NCHUNKS // NBUF, lambda i, _: body(i * NBUF, _), 0)

    for b in range(NBUF):
        c = NCHUNKS - NBUF + b
        row0 = base_row + c * CH
        pltpu.make_async_copy(o_v[b], out_hbm.at[pl.ds(row0, CH)],
                              sem_o[b]).wait()


@jax.jit
def _emb_ln(x, ids, tab, gamma, beta):
    mesh = plsc.VectorSubcoreMesh(core_axis_name="c", subcore_axis_name="s")
    return pl.kernel(
        _emb_ln_body,
        out_type=jax.ShapeDtypeStruct((N, HID), jnp.float32),
        mesh=mesh,
        compiler_params=pltpu.CompilerParams(needs_layout_passes=False),
        scratch_types=[
            pltpu.VMEM((CH,), jnp.int32),
            pltpu.VMEM((CH,), jnp.int32),
            pltpu.VMEM((CH, HID), jnp.float32),
            pltpu.VMEM((CH, HID), jnp.float32),
            pltpu.VMEM((CH, HID), jnp.float32),
            pltpu.VMEM((CH, HID), jnp.float32),
            pltpu.VMEM((CH, HID), jnp.float32),
            pltpu.VMEM((CH, HID), jnp.float32),
            pltpu.VMEM((HID,), jnp.float32),
            pltpu.VMEM((HID,), jnp.float32),
            pltpu.VMEM((CH * 2 * LANES,), jnp.float32),
            pltpu.SMEM((CH,), jnp.float32),
            pltpu.SMEM((CH,), jnp.float32),
            pltpu.SemaphoreType.DMA,
            pltpu.SemaphoreType.DMA,
            pltpu.SemaphoreType.DMA,
            pltpu.SemaphoreType.DMA,
            pltpu.SemaphoreType.DMA,
            pltpu.SemaphoreType.DMA,
        ],
    )(x, ids, tab, gamma, beta)


def kernel(input_embeds, position_ids, pos_table, gamma, beta):
    x = input_embeds.reshape(N, HID)
    ids = position_ids.reshape(N)
    out = _emb_ln(x, ids, pos_table, gamma, beta)
    return out.reshape(B, S, HID)


# speedup vs baseline: 1.5716x; 1.0617x over previous
import functools

import jax
import jax.numpy as jnp
from jax import lax
from jax.experimental import pallas as pl
from jax.experimental.pallas import tpu as pltpu
from jax.experimental.pallas import tpu_sc as plsc

B, S, HID, MAXP = 4, 4096, 1024, 4096
EPS = 1e-12
N = B * S

LANES = 16
VPR = HID // LANES
CH = 16
NC, NS = 2, 16
NW = NC * NS
ROWS_PER_W = N // NW
NCHUNKS = ROWS_PER_W // CH
NBUF = 2
NACC = 4
STATS_OFF = CH * 2 * LANES


def _rsqrt(y):
    i = lax.bitcast_convert_type(y, jnp.int32)
    i = jnp.int32(0x5F3759DF) - lax.shift_right_arithmetic(i, 1)
    r = lax.bitcast_convert_type(i, jnp.float32)
    half = 0.5 * y
    for _ in range(3):
        r = r * (1.5 - half * r * r)
    return r


def _emb_ln_body(x_hbm, ids_hbm, tab_hbm, g_hbm, b_hbm, out_hbm,
                 idx0, idx1, x0, x1, t0, t1, o0, o1, g_v, b_v, stage_v,
                 sg0, sg1, sx0, sx1, so0, so1):
    idx_v = [idx0, idx1]
    x_v = [x0, x1]
    t_v = [t0, t1]
    o_v = [o0, o1]
    sem_g = [sg0, sg1]
    sem_x = [sx0, sx1]
    sem_o = [so0, so1]

    wid = lax.axis_index("s") * NC + lax.axis_index("c")
    base_row = wid * ROWS_PER_W

    pltpu.sync_copy(g_hbm, g_v)
    pltpu.sync_copy(b_hbm, b_v)

    zeros = jnp.zeros((LANES,), jnp.float32)

    def start_loads(c, bslot):
        row0 = base_row + c * CH
        pltpu.sync_copy(ids_hbm.at[pl.ds(row0, CH)], idx_v[bslot])
        pltpu.async_copy(tab_hbm.at[idx_v[bslot]], t_v[bslot], sem_g[bslot])
        pltpu.async_copy(x_hbm.at[pl.ds(row0, CH)], x_v[bslot], sem_x[bslot])

    lane_iota = jnp.arange(LANES, dtype=jnp.int32)
    col_base = lane_iota * (2 * LANES)

    def compute(bslot):
        xb, tb, ob = x_v[bslot], t_v[bslot], o_v[bslot]

        def row_stats(r, _):
            def acc(g, carry):
                accs = list(carry)
                base = g * (NACC * LANES)
                for k in range(NACC):
                    sl = pl.ds(base + k * LANES, LANES)
                    v = xb[r, sl] + tb[r, sl]
                    xb[r, sl] = v
                    accs[k] = accs[k] + v
                    accs[NACC + k] = accs[NACC + k] + v * v
                return tuple(accs)

            accs = lax.fori_loop(0, VPR // NACC, acc, (zeros,) * (2 * NACC),
                                 unroll=4)
            s = (accs[0] + accs[1]) + (accs[2] + accs[3])
            q = (accs[4] + accs[5]) + (accs[6] + accs[7])
            stage_v[pl.ds(r * 2 * LANES, LANES)] = s
            stage_v[pl.ds(r * 2 * LANES + LANES, LANES)] = q
            return 0

        lax.fori_loop(0, CH, row_stats, 0)

        srow = zeros
        qrow = zeros
        for l in range(LANES):
            srow = srow + plsc.load_gather(stage_v, [col_base + l])
            qrow = qrow + plsc.load_gather(stage_v, [col_base + LANES + l])
        mean_v = srow * (1.0 / HID)
        var_v = qrow * (1.0 / HID) - mean_v * mean_v
        rstd_v = _rsqrt(var_v + EPS)
        stage_v[pl.ds(STATS_OFF, LANES)] = mean_v
        stage_v[pl.ds(STATS_OFF + LANES, LANES)] = rstd_v

        bm = []
        br = []
        for r in range(CH):
            splat = jnp.full((LANES,), r, dtype=jnp.int32)
            bm.append(plsc.load_gather(stage_v, [STATS_OFF + splat]))
            br.append(plsc.load_gather(stage_v, [STATS_OFF + LANES + splat]))

        def col(j, _):
            sl = pl.ds(j * LANES, LANES)
            g = g_v[sl]
            b = b_v[sl]
            for r in range(CH):
                v = xb[r, sl]
                ob[r, sl] = (v - bm[r]) * br[r] * g + b
            return 0

        lax.fori_loop(0, VPR, col, 0)

    for b in range(NBUF):
        start_loads(b, b)

    def body(c0, _):
        for b in range(NBUF):
            c = c0 + b
            pltpu.make_async_copy(tab_hbm.at[idx_v[b]], t_v[b], sem_g[b]).wait()
            row0 = base_row + c * CH
            pltpu.make_async_copy(x_hbm.at[pl.ds(row0, CH)], x_v[b],
                                  sem_x[b]).wait()

            @pl.when(c >= NBUF)
            def _():
                prev0 = base_row + (c - NBUF) * CH
                pltpu.make_async_copy(o_v[b], out_hbm.at[pl.ds(prev0, CH)],
                                      sem_o[b]).wait()

            compute(b)
            pltpu.async_copy(o_v[b], out_hbm.at[pl.ds(row0, CH)], sem_o[b])

            @pl.when(c + NBUF < NCHUNKS)
            def _():
                start_loads(c + NBUF, b)

        return 0

    lax.fori_loop(0, NCHUNKS // NBUF, lambda i, _: body(i * NBUF, _), 0)

    for b in range(NBUF):
        c = NCHUNKS - NBUF + b
        row0 = base_row + c * CH
        pltpu.make_async_copy(o_v[b], out_hbm.at[pl.ds(row0, CH)],
                              sem_o[b]).wait()


@jax.jit
def _emb_ln(x, ids, tab, gamma, beta):
    mesh = plsc.VectorSubcoreMesh(core_axis_name="c", subcore_axis_name="s")
    return pl.kernel(
        _emb_ln_body,
        out_type=jax.ShapeDtypeStruct((N, HID), jnp.float32),
        mesh=mesh,
        compiler_params=pltpu.CompilerParams(needs_layout_passes=False),
        scratch_types=[
            pltpu.VMEM((CH,), jnp.int32),
            pltpu.VMEM((CH,), jnp.int32),
            pltpu.VMEM((CH, HID), jnp.float32),
            pltpu.VMEM((CH, HID), jnp.float32),
            pltpu.VMEM((CH, HID), jnp.float32),
            pltpu.VMEM((CH, HID), jnp.float32),
            pltpu.VMEM((CH, HID), jnp.float32),
            pltpu.VMEM((CH, HID), jnp.float32),
            pltpu.VMEM((HID,), jnp.float32),
            pltpu.VMEM((HID,), jnp.float32),
            pltpu.VMEM(((CH * 2 + 2) * LANES,), jnp.float32),
            pltpu.SemaphoreType.DMA,
            pltpu.SemaphoreType.DMA,
            pltpu.SemaphoreType.DMA,
            pltpu.SemaphoreType.DMA,
            pltpu.SemaphoreType.DMA,
            pltpu.SemaphoreType.DMA,
        ],
    )(x, ids, tab, gamma, beta)


def kernel(input_embeds, position_ids, pos_table, gamma, beta):
    x = input_embeds.reshape(N, HID)
    ids = position_ids.reshape(N)
    out = _emb_ln(x, ids, pos_table, gamma, beta)
    return out.reshape(B, S, HID)


# speedup vs baseline: 3.5747x; 2.2746x over previous
import functools

import jax
import jax.numpy as jnp
from jax import lax
from jax.experimental import pallas as pl
from jax.experimental.pallas import tpu as pltpu
from jax.experimental.pallas import tpu_sc as plsc

B, S, HID, MAXP = 4, 4096, 1024, 4096
EPS = 1e-12
N = B * S

LANES = 16
VPR = HID // LANES
CH = 16
NC, NS = 2, 16
NW = NC * NS
ROWS_PER_W = N // NW
NCHUNKS = ROWS_PER_W // CH
NBUF = 2
NACC = 4
STATS_OFF = CH * 2 * LANES


def _rsqrt(y):
    i = lax.bitcast_convert_type(y, jnp.int32)
    i = jnp.int32(0x5F3759DF) - lax.shift_right_arithmetic(i, 1)
    r = lax.bitcast_convert_type(i, jnp.float32)
    half = 0.5 * y
    for _ in range(3):
        r = r * (1.5 - half * r * r)
    return r


def _emb_ln_body(x_hbm, ids_hbm, tab_hbm, g_hbm, b_hbm, out_hbm,
                 idx0, idx1, x0, x1, t0, t1, o0, o1, g_v, b_v, stage_v,
                 sg0, sg1, sx0, sx1, so0, so1):
    idx_v = [idx0, idx1]
    x_v = [x0, x1]
    t_v = [t0, t1]
    o_v = [o0, o1]
    sem_g = [sg0, sg1]
    sem_x = [sx0, sx1]
    sem_o = [so0, so1]

    wid = lax.axis_index("s") * NC + lax.axis_index("c")
    base_row = wid * ROWS_PER_W

    pltpu.sync_copy(g_hbm, g_v)
    pltpu.sync_copy(b_hbm, b_v)

    zeros = jnp.zeros((LANES,), jnp.float32)

    def start_loads(c, bslot):
        row0 = base_row + c * CH
        pltpu.sync_copy(ids_hbm.at[pl.ds(row0, CH)], idx_v[bslot])
        pltpu.async_copy(tab_hbm.at[idx_v[bslot]], t_v[bslot], sem_g[bslot])
        pltpu.async_copy(x_hbm.at[pl.ds(row0, CH)], x_v[bslot], sem_x[bslot])

    lane_iota = jnp.arange(LANES, dtype=jnp.int32)
    col_base = lane_iota * (2 * LANES)

    def compute(bslot):
        xb, tb, ob = x_v[bslot], t_v[bslot], o_v[bslot]

        def row_stats(r, _):
            def acc(j, carry):
                accs = list(carry)
                for k in range(NACC):
                    sl = pl.ds((j + k) * LANES, LANES)
                    v = xb[r, sl] + tb[r, sl]
                    xb[r, sl] = v
                    accs[k] = accs[k] + v
                    accs[NACC + k] = accs[NACC + k] + v * v
                return tuple(accs)

            accs = plsc.parallel_loop(
                0, VPR, step=NACC, unroll=2,
                carry=(zeros,) * (2 * NACC))(acc)
            s = (accs[0] + accs[1]) + (accs[2] + accs[3])
            q = (accs[4] + accs[5]) + (accs[6] + accs[7])
            stage_v[pl.ds(r * 2 * LANES, LANES)] = s
            stage_v[pl.ds(r * 2 * LANES + LANES, LANES)] = q
            return 0

        lax.fori_loop(0, CH, row_stats, 0)

        srow = zeros
        qrow = zeros
        for l in range(LANES):
            srow = srow + plsc.load_gather(stage_v, [col_base + l])
            qrow = qrow + plsc.load_gather(stage_v, [col_base + LANES + l])
        mean_v = srow * (1.0 / HID)
        var_v = qrow * (1.0 / HID) - mean_v * mean_v
        rstd_v = _rsqrt(var_v + EPS)
        stage_v[pl.ds(STATS_OFF, LANES)] = mean_v
        stage_v[pl.ds(STATS_OFF + LANES, LANES)] = rstd_v

        bm = []
        br = []
        for r in range(CH):
            splat = jnp.full((LANES,), r, dtype=jnp.int32)
            bm.append(plsc.load_gather(stage_v, [STATS_OFF + splat]))
            br.append(plsc.load_gather(stage_v, [STATS_OFF + LANES + splat]))

        def col(j):
            sl = pl.ds(j * LANES, LANES)
            g = g_v[sl]
            b = b_v[sl]
            for r in range(CH):
                v = xb[r, sl]
                ob[r, sl] = (v - bm[r]) * br[r] * g + b

        plsc.parallel_loop(0, VPR, step=1, unroll=1)(col)

    for b in range(NBUF):
        start_loads(b, b)

    def body(c0, _):
        for b in range(NBUF):
            c = c0 + b
            pltpu.make_async_copy(tab_hbm.at[idx_v[b]], t_v[b], sem_g[b]).wait()
            row0 = base_row + c * CH
            pltpu.make_async_copy(x_hbm.at[pl.ds(row0, CH)], x_v[b],
                                  sem_x[b]).wait()

            @pl.when(c >= NBUF)
            def _():
                prev0 = base_row + (c - NBUF) * CH
                pltpu.make_async_copy(o_v[b], out_hbm.at[pl.ds(prev0, CH)],
                                      sem_o[b]).wait()

            compute(b)
            pltpu.async_copy(o_v[b], out_hbm.at[pl.ds(row0, CH)], sem_o[b])

            @pl.when(c + NBUF < NCHUNKS)
            def _():
                start_loads(c + NBUF, b)

        return 0

    lax.fori_loop(0, NCHUNKS // NBUF, lambda i, _: body(i * NBUF, _), 0)

    for b in range(NBUF):
        c = NCHUNKS - NBUF + b
        row0 = base_row + c * CH
        pltpu.make_async_copy(o_v[b], out_hbm.at[pl.ds(row0, CH)],
                              sem_o[b]).wait()


@jax.jit
def _emb_ln(x, ids, tab, gamma, beta):
    mesh = plsc.VectorSubcoreMesh(core_axis_name="c", subcore_axis_name="s")
    return pl.kernel(
        _emb_ln_body,
        out_type=jax.ShapeDtypeStruct((N, HID), jnp.float32),
        mesh=mesh,
        compiler_params=pltpu.CompilerParams(needs_layout_passes=False),
        scratch_types=[
            pltpu.VMEM((CH,), jnp.int32),
            pltpu.VMEM((CH,), jnp.int32),
            pltpu.VMEM((CH, HID), jnp.float32),
            pltpu.VMEM((CH, HID), jnp.float32),
            pltpu.VMEM((CH, HID), jnp.float32),
            pltpu.VMEM((CH, HID), jnp.float32),
            pltpu.VMEM((CH, HID), jnp.float32),
            pltpu.VMEM((CH, HID), jnp.float32),
            pltpu.VMEM((HID,), jnp.float32),
            pltpu.VMEM((HID,), jnp.float32),
            pltpu.VMEM(((CH * 2 + 2) * LANES,), jnp.float32),
            pltpu.SemaphoreType.DMA,
            pltpu.SemaphoreType.DMA,
            pltpu.SemaphoreType.DMA,
            pltpu.SemaphoreType.DMA,
            pltpu.SemaphoreType.DMA,
            pltpu.SemaphoreType.DMA,
        ],
    )(x, ids, tab, gamma, beta)


def kernel(input_embeds, position_ids, pos_table, gamma, beta):
    x = input_embeds.reshape(N, HID)
    ids = position_ids.reshape(N)
    out = _emb_ln(x, ids, pos_table, gamma, beta)
    return out.reshape(B, S, HID)


# speedup vs baseline: 4.0156x; 1.1233x over previous
import functools

import jax
import jax.numpy as jnp
from jax import lax
from jax.experimental import pallas as pl
from jax.experimental.pallas import tpu as pltpu
from jax.experimental.pallas import tpu_sc as plsc

B, S, HID, MAXP = 4, 4096, 1024, 4096
EPS = 1e-12
N = B * S

LANES = 16
VPR = HID // LANES
CH = 16
NC, NS = 2, 16
NW = NC * NS
ROWS_PER_W = N // NW
NCHUNKS = ROWS_PER_W // CH
NBUF = 2
NACC = 4
STATS_OFF = CH * 2 * LANES


def _rsqrt(y):
    i = lax.bitcast_convert_type(y, jnp.int32)
    i = jnp.int32(0x5F3759DF) - lax.shift_right_arithmetic(i, 1)
    r = lax.bitcast_convert_type(i, jnp.float32)
    half = 0.5 * y
    for _ in range(3):
        r = r * (1.5 - half * r * r)
    return r


def _emb_ln_body(x_hbm, ids_hbm, tab_hbm, g_hbm, b_hbm, out_hbm,
                 idxall_v, x0, x1, t0, t1, o0, o1, g_v, b_v, stage_v,
                 sg0, sg1, sx0, sx1, so0, so1):
    x_v = [x0, x1]
    t_v = [t0, t1]
    o_v = [o0, o1]
    sem_g = [sg0, sg1]
    sem_x = [sx0, sx1]
    sem_o = [so0, so1]

    wid = lax.axis_index("s") * NC + lax.axis_index("c")
    base_row = wid * ROWS_PER_W

    pltpu.sync_copy(g_hbm, g_v)
    pltpu.sync_copy(b_hbm, b_v)
    pltpu.sync_copy(ids_hbm.at[pl.ds(base_row, ROWS_PER_W)], idxall_v)

    zeros = jnp.zeros((LANES,), jnp.float32)

    def start_loads(c, bslot):
        row0 = base_row + c * CH
        pltpu.async_copy(tab_hbm.at[idxall_v.at[pl.ds(c * CH, CH)]],
                         t_v[bslot], sem_g[bslot])
        pltpu.async_copy(x_hbm.at[pl.ds(row0, CH)], x_v[bslot], sem_x[bslot])

    lane_iota = jnp.arange(LANES, dtype=jnp.int32)
    col_base = lane_iota * (2 * LANES)

    def pass1(bslot):
        xb, tb, ob = x_v[bslot], t_v[bslot], o_v[bslot]

        def row_stats(r, _):
            def acc(j, carry):
                accs = list(carry)
                for k in range(NACC):
                    sl = pl.ds((j + k) * LANES, LANES)
                    v = xb[r, sl] + tb[r, sl]
                    ob[r, sl] = v
                    accs[k] = accs[k] + v
                    accs[NACC + k] = accs[NACC + k] + v * v
                return tuple(accs)

            accs = plsc.parallel_loop(
                0, VPR, step=NACC, unroll=2,
                carry=(zeros,) * (2 * NACC))(acc)
            s = (accs[0] + accs[1]) + (accs[2] + accs[3])
            q = (accs[4] + accs[5]) + (accs[6] + accs[7])
            stage_v[pl.ds(r * 2 * LANES, LANES)] = s
            stage_v[pl.ds(r * 2 * LANES + LANES, LANES)] = q
            return 0

        lax.fori_loop(0, CH, row_stats, 0)

    def pass2(bslot):
        ob = o_v[bslot]

        srow = zeros
        qrow = zeros
        for l in range(LANES):
            srow = srow + plsc.load_gather(stage_v, [col_base + l])
            qrow = qrow + plsc.load_gather(stage_v, [col_base + LANES + l])
        mean_v = srow * (1.0 / HID)
        var_v = qrow * (1.0 / HID) - mean_v * mean_v
        rstd_v = _rsqrt(var_v + EPS)
        stage_v[pl.ds(STATS_OFF, LANES)] = mean_v
        stage_v[pl.ds(STATS_OFF + LANES, LANES)] = rstd_v

        bm = []
        br = []
        for r in range(CH):
            splat = jnp.full((LANES,), r, dtype=jnp.int32)
            bm.append(plsc.load_gather(stage_v, [STATS_OFF + splat]))
            br.append(plsc.load_gather(stage_v, [STATS_OFF + LANES + splat]))

        def col(j):
            sl = pl.ds(j * LANES, LANES)
            g = g_v[sl]
            b = b_v[sl]
            for r in range(CH):
                v = ob[r, sl]
                ob[r, sl] = (v - bm[r]) * br[r] * g + b

        plsc.parallel_loop(0, VPR, step=1, unroll=1)(col)

    for b in range(NBUF):
        start_loads(b, b)

    def body(c0, _):
        for b in range(NBUF):
            c = c0 + b
            row0 = base_row + c * CH
            pltpu.make_async_copy(
                tab_hbm.at[idxall_v.at[pl.ds(c * CH, CH)]], t_v[b],
                sem_g[b]).wait()
            pltpu.make_async_copy(x_hbm.at[pl.ds(row0, CH)], x_v[b],
                                  sem_x[b]).wait()

            @pl.when(c >= NBUF)
            def _():
                prev0 = base_row + (c - NBUF) * CH
                pltpu.make_async_copy(o_v[b], out_hbm.at[pl.ds(prev0, CH)],
                                      sem_o[b]).wait()

            pass1(b)

            @pl.when(c + NBUF < NCHUNKS)
            def _():
                start_loads(c + NBUF, b)

            pass2(b)
            pltpu.async_copy(o_v[b], out_hbm.at[pl.ds(row0, CH)], sem_o[b])

        return 0

    lax.fori_loop(0, NCHUNKS // NBUF, lambda i, _: body(i * NBUF, _), 0)

    for b in range(NBUF):
        c = NCHUNKS - NBUF + b
        row0 = base_row + c * CH
        pltpu.make_async_copy(o_v[b], out_hbm.at[pl.ds(row0, CH)],
                              sem_o[b]).wait()


@jax.jit
def _emb_ln(x, ids, tab, gamma, beta):
    mesh = plsc.VectorSubcoreMesh(core_axis_name="c", subcore_axis_name="s")
    return pl.kernel(
        _emb_ln_body,
        out_type=jax.ShapeDtypeStruct((N, HID), jnp.float32),
        mesh=mesh,
        compiler_params=pltpu.CompilerParams(needs_layout_passes=False),
        scratch_types=[
            pltpu.VMEM((ROWS_PER_W,), jnp.int32),
            pltpu.VMEM((CH, HID), jnp.float32),
            pltpu.VMEM((CH, HID), jnp.float32),
            pltpu.VMEM((CH, HID), jnp.float32),
            pltpu.VMEM((CH, HID), jnp.float32),
            pltpu.VMEM((CH, HID), jnp.float32),
            pltpu.VMEM((CH, HID), jnp.float32),
            pltpu.VMEM((HID,), jnp.float32),
            pltpu.VMEM((HID,), jnp.float32),
            pltpu.VMEM(((CH * 2 + 2) * LANES,), jnp.float32),
            pltpu.SemaphoreType.DMA,
            pltpu.SemaphoreType.DMA,
            pltpu.SemaphoreType.DMA,
            pltpu.SemaphoreType.DMA,
            pltpu.SemaphoreType.DMA,
            pltpu.SemaphoreType.DMA,
        ],
    )(x, ids, tab, gamma, beta)


def kernel(input_embeds, position_ids, pos_table, gamma, beta):
    x = input_embeds.reshape(N, HID)
    ids = position_ids.reshape(N)
    out = _emb_ln(x, ids, pos_table, gamma, beta)
    return out.reshape(B, S, HID)
